# Initial kernel scaffold; baseline (speedup 1.0000x reference)
import jax
import jax.numpy as jnp
from jax.experimental import pallas as pl


def kernel(position_matrix, channel_matrix, indices, coefficients):
    raise NotImplementedError("write your pallas kernel here")


# speedup vs baseline: 1.4953x; 1.4953x over previous
import functools
import jax
import jax.numpy as jnp
import numpy as np
from jax import lax
from jax.experimental import pallas as pl
from jax.experimental.pallas import tpu as pltpu
from jax.experimental.pallas import tpu_sc as plsc

N = 10000
K = 32
CIN = 128
COUT = 128
NDEG = 3
LDEG = 2
MDEG = 2
NB = NDEG * LDEG * MDEG
EPS = 1e-8

QB1 = 400
QB2 = 200

_INTERPRET = False


def _knn_body(posq_ref, post_ref, idx_ref, d2_s, sel_s):
    qx = posq_ref[:, 0:1]
    qy = posq_ref[:, 1:2]
    qz = posq_ref[:, 2:3]
    px = post_ref[0:1, :]
    py = post_ref[1:2, :]
    pz = post_ref[2:3, :]
    dx = qx - px
    dy = qy - py
    dz = qz - pz
    d2_s[...] = (dx * dx + dy * dy) + dz * dz

    iota = lax.broadcasted_iota(jnp.int32, (1, N), 1)
    cols = []
    for _ in range(K):
        d2 = d2_s[...]
        m = jnp.min(d2, axis=1, keepdims=True)
        cand = jnp.where(d2 == m, iota, N)
        am = jnp.min(cand, axis=1, keepdims=True)
        cols.append(am)
        onehot = iota == am
        d2_s[...] = jnp.where(onehot, jnp.inf, d2)
    idx_ref[...] = jnp.concatenate(cols, axis=1)


def _knn(pos8, pos_t):
    return pl.pallas_call(
        _knn_body,
        grid=(N // QB1,),
        in_specs=[
            pl.BlockSpec((QB1, 8), lambda i: (i, 0)),
            pl.BlockSpec((8, N), lambda i: (0, 0)),
        ],
        out_specs=pl.BlockSpec((QB1, K), lambda i: (i, 0)),
        out_shape=jax.ShapeDtypeStruct((N, K), jnp.int32),
        scratch_shapes=[
            pltpu.VMEM((QB1, N), jnp.float32),
            pltpu.VMEM((QB1, N), jnp.float32),
        ],
        interpret=_INTERPRET,
    )(pos8, pos_t)


def _sc_gather_pos(posx, posy, posz, flat_idx, chunk=80):
    info = plsc.get_sparse_core_info()
    nc, ns = info.num_cores, info.num_subcores
    nw = nc * ns
    m = flat_idx.shape[0]
    per_w = m // nw
    n_ch = per_w // chunk
    n_g = chunk // 16

    @functools.partial(
        pl.kernel,
        out_type=(jax.ShapeDtypeStruct((m,), jnp.float32),) * 3,
        mesh=plsc.VectorSubcoreMesh(core_axis_name="c", subcore_axis_name="s"),
        compiler_params=pltpu.CompilerParams(needs_layout_passes=False),
        scratch_types=[
            pltpu.VMEM((N,), jnp.float32),
            pltpu.VMEM((N,), jnp.float32),
            pltpu.VMEM((N,), jnp.float32),
            pltpu.VMEM((chunk,), jnp.int32),
            pltpu.VMEM((chunk,), jnp.float32),
            pltpu.VMEM((chunk,), jnp.float32),
            pltpu.VMEM((chunk,), jnp.float32),
        ],
    )
    def gather_k(px_hbm, py_hbm, pz_hbm, idx_hbm, ox_hbm, oy_hbm, oz_hbm,
                 px_v, py_v, pz_v, idx_v, bx_v, by_v, bz_v):
        wid = lax.axis_index("s") * nc + lax.axis_index("c")
        base = wid * per_w
        pltpu.sync_copy(px_hbm, px_v)
        pltpu.sync_copy(py_hbm, py_v)
        pltpu.sync_copy(pz_hbm, pz_v)

        def body(c, carry):
            off = base + c * chunk
            pltpu.sync_copy(idx_hbm.at[pl.ds(off, chunk)], idx_v)
            for g in range(n_g):
                i16 = idx_v[pl.ds(g * 16, 16)]
                bx_v[pl.ds(g * 16, 16)] = plsc.load_gather(px_v, [i16])
                by_v[pl.ds(g * 16, 16)] = plsc.load_gather(py_v, [i16])
                bz_v[pl.ds(g * 16, 16)] = plsc.load_gather(pz_v, [i16])
            pltpu.sync_copy(bx_v, ox_hbm.at[pl.ds(off, chunk)])
            pltpu.sync_copy(by_v, oy_hbm.at[pl.ds(off, chunk)])
            pltpu.sync_copy(bz_v, oz_hbm.at[pl.ds(off, chunk)])
            return carry

        lax.fori_loop(0, n_ch, body, 0)

    return gather_k(posx, posy, posz, flat_idx)


def _sc_gather(table, flat_idx, width, chunk):
    info = plsc.get_sparse_core_info()
    nc, ns = info.num_cores, info.num_subcores
    nw = nc * ns
    m = flat_idx.shape[0]
    per_w = m // nw
    n_ch = per_w // chunk

    @functools.partial(
        pl.kernel,
        out_type=jax.ShapeDtypeStruct((m, width), jnp.float32),
        mesh=plsc.VectorSubcoreMesh(core_axis_name="c", subcore_axis_name="s"),
        scratch_types=[
            pltpu.VMEM((chunk,), jnp.int32),
            pltpu.VMEM((chunk, width), jnp.float32),
            pltpu.SemaphoreType.DMA,
        ],
    )
    def gather_k(table_hbm, idx_hbm, out_hbm, idx_v, rows_v, sem):
        wid = lax.axis_index("s") * nc + lax.axis_index("c")
        base = wid * per_w

        def body(c, carry):
            off = base + c * chunk
            pltpu.sync_copy(idx_hbm.at[pl.ds(off, chunk)], idx_v)
            pltpu.async_copy(table_hbm.at[idx_v], rows_v, sem).wait()
            pltpu.sync_copy(rows_v, out_hbm.at[pl.ds(off, chunk)])
            return carry

        lax.fori_loop(0, n_ch, body, 0)

    return gather_k(table, flat_idx)


def _cen_body(gx_ref, gy_ref, gz_ref, cen_ref):
    c0 = jnp.mean(gx_ref[...], axis=1, keepdims=True)
    c1 = jnp.mean(gy_ref[...], axis=1, keepdims=True)
    c2 = jnp.mean(gz_ref[...], axis=1, keepdims=True)
    cen_ref[...] = jnp.concatenate(
        [c0, c1, c2, jnp.zeros((cen_ref.shape[0], 13), jnp.float32)], axis=1)


def _cen(gx, gy, gz):
    return pl.pallas_call(
        _cen_body,
        grid=(N // QB2,),
        in_specs=[pl.BlockSpec((QB2, K), lambda i: (i, 0))] * 3,
        out_specs=pl.BlockSpec((QB2, 16), lambda i: (i, 0)),
        out_shape=jax.ShapeDtypeStruct((N, 16), jnp.float32),
        interpret=_INTERPRET,
    )(gx, gy, gz)


def _agg_body(gx_ref, gy_ref, gz_ref, gfeat_ref, cen_ref, evec_ref, coef_ref,
              out_ref):
    p0 = gx_ref[...]
    p1 = gy_ref[...]
    p2 = gz_ref[...]
    l0 = p0 - cen_ref[:, 0:1]
    l1 = p1 - cen_ref[:, 1:2]
    l2 = p2 - cen_ref[:, 2:3]

    def e(i, j):
        return evec_ref[:, 3 * i + j:3 * i + j + 1]

    x = l0 * e(0, 0) + l1 * e(1, 0) + l2 * e(2, 0)
    y = l0 * e(0, 1) + l1 * e(1, 1) + l2 * e(2, 1)
    z = l0 * e(0, 2) + l1 * e(1, 2) + l2 * e(2, 2)

    r = jnp.sqrt((x * x + y * y) + z * z)
    ct = jnp.clip(z / (r + EPS), -1.0, 1.0)
    hxy = jnp.sqrt(x * x + y * y)
    cp = jnp.where(hxy > 0.0, x / hxy, 1.0)
    rmax = jnp.max(r, axis=1, keepdims=True) + EPS
    rn = r / rmax

    rad = [None, rn, rn * rn]
    ang_t = [None, ct]
    ang_p = [None, cp]
    acc = jnp.zeros((out_ref.shape[0], COUT), jnp.float32)
    b = 0
    for a in range(NDEG):
        for l in range(LDEG):
            for mm in range(MDEG):
                w = None
                for f in (rad[a], ang_t[l], ang_p[mm]):
                    if f is not None:
                        w = f if w is None else w * f
                if w is None:
                    t = gfeat_ref[:, 0, :]
                    for k in range(1, K):
                        t = t + gfeat_ref[:, k, :]
                else:
                    t = w[:, 0:1] * gfeat_ref[:, 0, :]
                    for k in range(1, K):
                        t = t + w[:, k:k + 1] * gfeat_ref[:, k, :]
                acc = acc + jnp.dot(t, coef_ref[b],
                                    preferred_element_type=jnp.float32)
                b += 1
    out_ref[...] = acc


def _agg(gx, gy, gz, gfeat, cen16, evec16, coef3):
    return pl.pallas_call(
        _agg_body,
        grid=(N // QB2,),
        in_specs=[
            pl.BlockSpec((QB2, K), lambda i: (i, 0)),
            pl.BlockSpec((QB2, K), lambda i: (i, 0)),
            pl.BlockSpec((QB2, K), lambda i: (i, 0)),
            pl.BlockSpec((QB2, K, CIN), lambda i: (i, 0, 0)),
            pl.BlockSpec((QB2, 16), lambda i: (i, 0)),
            pl.BlockSpec((QB2, 16), lambda i: (i, 0)),
            pl.BlockSpec((NB, CIN, COUT), lambda i: (0, 0, 0)),
        ],
        out_specs=pl.BlockSpec((QB2, COUT), lambda i: (i, 0)),
        out_shape=jax.ShapeDtypeStruct((N, COUT), jnp.float32),
        interpret=_INTERPRET,
    )(gx, gy, gz, gfeat, cen16, evec16, coef3)


def kernel(position_matrix, channel_matrix, indices, coefficients):
    del indices
    pos8 = jnp.pad(position_matrix, ((0, 0), (0, 5)))
    pos_t = jnp.pad(position_matrix.T, ((0, 5), (0, 0)))
    coef3 = jnp.transpose(coefficients, (2, 1, 0))

    nbr = _knn(pos8, pos_t)
    flat_idx = nbr.reshape(-1)

    gx, gy, gz = _sc_gather_pos(pos_t[0], pos_t[1], pos_t[2], flat_idx)
    gx = gx.reshape(N, K)
    gy = gy.reshape(N, K)
    gz = gz.reshape(N, K)
    gfeat = _sc_gather(channel_matrix, flat_idx, CIN, 80).reshape(N, K, CIN)

    cen16 = _cen(gx, gy, gz)

    nbr_pos = jnp.stack([gx, gy, gz], axis=-1)
    centers_j = jnp.mean(nbr_pos, axis=1)
    local = nbr_pos - centers_j[:, None, :]
    cov = jnp.einsum('nki,nkj->nij', local, local) / float(K)
    _, evecs = jnp.linalg.eigh(cov)
    evec16 = jnp.pad(evecs.reshape(N, 9), ((0, 0), (0, 7)))

    out = _agg(gx, gy, gz, gfeat, cen16, evec16, coef3)
    centers = cen16[:, :3]
    return (centers, out)
